# baseline (speedup 1.0000x reference)
import jax
import jax.numpy as jnp
from jax import lax
from jax.experimental import pallas as pl
from jax.experimental.pallas import tpu as pltpu
from jax.experimental.pallas import tpu_sc as plsc

B, P, H, W, D, C = 4, 8, 32, 32, 32, 32
SLABS = B * P
NPTS = H * W * D
NSC = 2
NTILE = 16
PPT = NPTS // NTILE
CH = 64
NPAIR = PPT // (2 * CH)
NGRP = CH // 16
NSTREAM = (8 * CH) // 128

_f32 = jnp.float32
_i32 = jnp.int32


def _resample_kernel(table, theta_p, lin, out, th_v, lin_v, idx_v, w_v,
                     rows_v, out_v, shared, sem0, sem1):
    cidx = lax.axis_index("c")
    sid = lax.axis_index("s")

    pltpu.sync_copy(lin.at[0], lin_v)
    sems = (sem0, sem1)
    lane = lax.iota(_i32, 16)

    def round_body(rnd, carry):
        slab = cidx * (SLABS // NSC) + rnd
        slab_base = slab * NPTS

        pltpu.sync_copy(table.at[pl.ds(slab_base + sid * PPT, PPT)],
                        shared.at[pl.ds(sid * PPT, PPT)])
        pltpu.sync_copy(theta_p.at[slab], th_v)
        plsc.subcore_barrier()

        th_vec = th_v[...]
        t = [th_vec[i] for i in range(12)]

        def compute_chunk(pbase, buf):
            for g in range(NGRP):
                pv = pbase + g * 16 + lane
                dv = pv & 31
                wv = (pv >> 5) & 31
                hv = pv >> 10
                xb = plsc.load_gather(lin_v, [wv])
                yb = plsc.load_gather(lin_v, [hv])
                zb = plsc.load_gather(lin_v, [dv])
                bs = []
                ts = []
                for ax in range(3):
                    T0, T1, T2, T3 = (t[4 * ax], t[4 * ax + 1],
                                      t[4 * ax + 2], t[4 * ax + 3])
                    sv = T0 * xb + T1 * yb + T2 * zb + T3
                    gv = _f32(0.5) * ((sv + _f32(1.0)) * _f32(30.0))
                    bi = jnp.clip(gv.astype(_i32), 0, 30)
                    bs.append(bi)
                    ts.append(jnp.clip(gv - bi.astype(_f32),
                                       _f32(0.0), _f32(1.0)))
                bx, by, bz = bs
                tx, ty, tz = ts
                ux = _f32(1.0) - tx
                uy = _f32(1.0) - ty
                uz = _f32(1.0) - tz
                base = (by << 10) + (bx << 5) + bz
                for k in range(8):
                    ix, jy, kz = (k >> 2) & 1, (k >> 1) & 1, k & 1
                    idx_k = base + jy * 1024 + ix * 32 + kz
                    w_k = ((tx if ix else ux) * (ty if jy else uy)
                           * (tz if kz else uz))
                    e = k * CH + g * 16
                    idx_v[buf, e >> 7, pl.ds(e & 127, 16)] = idx_k
                    w_v[buf, k, pl.ds(g * 16, 16)] = w_k

        def fire(buf):
            return [pltpu.async_copy(shared.at[idx_v.at[buf, j]],
                                     rows_v.at[buf, pl.ds(j * 128, 128)],
                                     sems[buf])
                    for j in range(NSTREAM)]

        def combine_store(pbase, buf):
            def grp_body(g2, c2):
                gbase = g2 * 16
                wvecs = [w_v[buf, k, pl.ds(gbase, 16)] for k in range(8)]
                for j in range(16):
                    p = gbase + j
                    ws = [wvecs[k][j] for k in range(8)]
                    for half, off in ((0, 0), (1, 16)):
                        m = [ws[k] * rows_v[buf, k * CH + p, pl.ds(off, 16)]
                             for k in range(8)]
                        a0 = m[0] + m[1]
                        a1 = m[2] + m[3]
                        a2 = m[4] + m[5]
                        a3 = m[6] + m[7]
                        out_v[p, pl.ds(off, 16)] = (a0 + a1) + (a2 + a3)
                return c2

            lax.fori_loop(0, NGRP, grp_body, 0)
            pltpu.sync_copy(out_v, out.at[pl.ds(slab_base + pbase, CH)])

        def pair_body(i2, c2):
            pbase = sid * PPT + i2 * 2 * CH
            compute_chunk(pbase, 0)
            h0 = fire(0)
            compute_chunk(pbase + CH, 1)
            h1 = fire(1)
            for h in h0:
                h.wait()
            combine_store(pbase, 0)
            for h in h1:
                h.wait()
            combine_store(pbase + CH, 1)
            return c2

        lax.fori_loop(0, NPAIR, pair_body, 0)
        plsc.subcore_barrier()
        return carry

    lax.fori_loop(0, SLABS // NSC, round_body, 0)


@jax.jit
def kernel(input_fmap, theta):
    table = input_fmap.reshape(SLABS * NPTS, C)
    theta_bf = theta.astype(jnp.bfloat16).astype(_f32)
    theta_p = jnp.pad(theta_bf.reshape(SLABS, 12), ((0, 0), (0, 4)))
    lin = jnp.linspace(-1.0, 1.0, 32).astype(jnp.bfloat16).astype(_f32)
    lin = lin.reshape(1, 32)
    mesh = plsc.VectorSubcoreMesh(core_axis_name="c", subcore_axis_name="s",
                                  num_cores=2, num_subcores=16)
    run = pl.kernel(
        _resample_kernel,
        out_type=jax.ShapeDtypeStruct((SLABS * NPTS, C), _f32),
        mesh=mesh,
        scratch_types=[
            pltpu.VMEM((16,), _f32),
            pltpu.VMEM((32,), _f32),
            pltpu.VMEM((2, NSTREAM, 128), _i32),
            pltpu.VMEM((2, 8, CH), _f32),
            pltpu.VMEM((2, 8 * CH, C), _f32),
            pltpu.VMEM((CH, C), _f32),
            pltpu.VMEM_SHARED((NPTS, C), _f32),
            pltpu.SemaphoreType.DMA,
            pltpu.SemaphoreType.DMA,
        ],
        compiler_params=pltpu.CompilerParams(use_tc_tiling_on_sc=False,
                                             needs_layout_passes=False),
    )
    out = run(table, theta_p, lin)
    return out.reshape(B, P, H, W, D, C)


# speedup vs baseline: 1.0196x; 1.0196x over previous
import jax
import jax.numpy as jnp
from jax import lax
from jax.experimental import pallas as pl
from jax.experimental.pallas import tpu as pltpu
from jax.experimental.pallas import tpu_sc as plsc

B, P, H, W, D, C = 4, 8, 32, 32, 32, 32
SLABS = B * P
NPTS = H * W * D
NSC = 2
NTILE = 16
PPT = NPTS // NTILE
CH = 64
NPAIR = PPT // (2 * CH)
NGRP = CH // 16
NSTREAM = (8 * CH) // 128

_f32 = jnp.float32
_i32 = jnp.int32


def _resample_kernel(table, theta_p, lin, out, th_v, lin_v, idx_v, w_v,
                     rows_v, out_v, shared, sem0, sem1):
    cidx = lax.axis_index("c")
    sid = lax.axis_index("s")

    pltpu.sync_copy(lin.at[0], lin_v)
    sems = (sem0, sem1)
    lane = lax.iota(_i32, 16)

    def round_body(rnd, carry):
        slab = cidx * (SLABS // NSC) + rnd
        slab_base = slab * NPTS

        pltpu.sync_copy(table.at[pl.ds(slab_base + sid * PPT, PPT)],
                        shared.at[pl.ds(sid * PPT, PPT)])
        pltpu.sync_copy(theta_p.at[slab], th_v)
        plsc.subcore_barrier()

        th_vec = th_v[...]
        t = [th_vec[i] for i in range(12)]

        def compute_chunk(pbase, buf):
            for g in range(NGRP):
                pv = pbase + g * 16 + lane
                dv = pv & 31
                wv = (pv >> 5) & 31
                hv = pv >> 10
                xb = plsc.load_gather(lin_v, [wv])
                yb = plsc.load_gather(lin_v, [hv])
                zb = plsc.load_gather(lin_v, [dv])
                bs = []
                ts = []
                for ax in range(3):
                    T0, T1, T2, T3 = (t[4 * ax], t[4 * ax + 1],
                                      t[4 * ax + 2], t[4 * ax + 3])
                    sv = T0 * xb + T1 * yb + T2 * zb + T3
                    gv = _f32(0.5) * ((sv + _f32(1.0)) * _f32(30.0))
                    bi = jnp.clip(gv.astype(_i32), 0, 30)
                    bs.append(bi)
                    ts.append(jnp.clip(gv - bi.astype(_f32),
                                       _f32(0.0), _f32(1.0)))
                bx, by, bz = bs
                tx, ty, tz = ts
                ux = _f32(1.0) - tx
                uy = _f32(1.0) - ty
                uz = _f32(1.0) - tz
                base = (by << 10) + (bx << 5) + bz
                for k in range(8):
                    ix, jy, kz = (k >> 2) & 1, (k >> 1) & 1, k & 1
                    idx_k = base + jy * 1024 + ix * 32 + kz
                    w_k = ((tx if ix else ux) * (ty if jy else uy)
                           * (tz if kz else uz))
                    e = k * CH + g * 16
                    idx_v[buf, e >> 7, pl.ds(e & 127, 16)] = idx_k
                    w_v[buf, k, pl.ds(g * 16, 16)] = w_k

        def fire(buf):
            return [pltpu.async_copy(shared.at[idx_v.at[buf, j]],
                                     rows_v.at[buf, pl.ds(j * 128, 128)],
                                     sems[buf])
                    for j in range(NSTREAM)]

        def combine_store(pbase, buf):
            def grp_body(g2, c2):
                gbase = g2 * 16
                wvecs = [w_v[buf, k, pl.ds(gbase, 16)] for k in range(8)]
                for j in range(16):
                    p = gbase + j
                    acc0 = jnp.zeros((16,), _f32)
                    acc1 = jnp.zeros((16,), _f32)
                    for k in range(8):
                        wk = wvecs[k][j]
                        r = k * CH + p
                        acc0 = acc0 + wk * rows_v[buf, r, pl.ds(0, 16)]
                        acc1 = acc1 + wk * rows_v[buf, r, pl.ds(16, 16)]
                    out_v[p, pl.ds(0, 16)] = acc0
                    out_v[p, pl.ds(16, 16)] = acc1
                return c2

            lax.fori_loop(0, NGRP, grp_body, 0)
            pltpu.sync_copy(out_v, out.at[pl.ds(slab_base + pbase, CH)])

        def pair_body(i2, c2):
            pbase = sid * PPT + i2 * 2 * CH
            compute_chunk(pbase, 0)
            h0 = fire(0)
            compute_chunk(pbase + CH, 1)
            h1 = fire(1)
            for h in h0:
                h.wait()
            combine_store(pbase, 0)
            for h in h1:
                h.wait()
            combine_store(pbase + CH, 1)
            return c2

        lax.fori_loop(0, NPAIR, pair_body, 0)
        plsc.subcore_barrier()
        return carry

    lax.fori_loop(0, SLABS // NSC, round_body, 0)


@jax.jit
def kernel(input_fmap, theta):
    table = input_fmap.reshape(SLABS * NPTS, C)
    theta_bf = theta.astype(jnp.bfloat16).astype(_f32)
    theta_p = jnp.pad(theta_bf.reshape(SLABS, 12), ((0, 0), (0, 4)))
    lin = jnp.linspace(-1.0, 1.0, 32).astype(jnp.bfloat16).astype(_f32)
    lin = lin.reshape(1, 32)
    mesh = plsc.VectorSubcoreMesh(core_axis_name="c", subcore_axis_name="s",
                                  num_cores=2, num_subcores=16)
    run = pl.kernel(
        _resample_kernel,
        out_type=jax.ShapeDtypeStruct((SLABS * NPTS, C), _f32),
        mesh=mesh,
        scratch_types=[
            pltpu.VMEM((16,), _f32),
            pltpu.VMEM((32,), _f32),
            pltpu.VMEM((2, NSTREAM, 128), _i32),
            pltpu.VMEM((2, 8, CH), _f32),
            pltpu.VMEM((2, 8 * CH, C), _f32),
            pltpu.VMEM((CH, C), _f32),
            pltpu.VMEM_SHARED((NPTS, C), _f32),
            pltpu.SemaphoreType.DMA,
            pltpu.SemaphoreType.DMA,
        ],
        compiler_params=pltpu.CompilerParams(use_tc_tiling_on_sc=False,
                                             needs_layout_passes=False),
    )
    out = run(table, theta_p, lin)
    return out.reshape(B, P, H, W, D, C)
